# baseline (speedup 1.0000x reference)
import functools

import jax
import jax.numpy as jnp
from jax import lax
from jax.experimental import pallas as pl
from jax.experimental.pallas import tpu as pltpu

IOU_THRESHOLD = 0.5
CONFIDENCE_THRESHOLD = 0.5
MAX_DETECTIONS = 100
NEG_INF = float("-inf")


def _nms_body(x1_ref, y1_ref, x2_ref, y2_ref, logits_ref,
              obx_ref, oby_ref, obw_ref, obh_ref, oconf_ref, ocls_ref,
              idx_scr, val_scr):
    B, N = x1_ref.shape
    x1 = x1_ref[...]
    y1 = y1_ref[...]
    w = x2_ref[...] - x1
    h = y2_ref[...] - y1
    a_min = jnp.minimum(x1, w)
    a_max = jnp.maximum(x1, w)
    b_min = jnp.minimum(y1, h)
    b_max = jnp.maximum(y1, h)
    areas = (a_max - a_min) * (b_max - b_min)

    parts = []
    for b in range(B):
        red = jnp.max(logits_ref[b], axis=-1, keepdims=True)
        parts.append(jnp.transpose(red, (1, 0)))
    conf = jax.nn.sigmoid(jnp.concatenate(parts, axis=0))

    iota = lax.broadcasted_iota(jnp.int32, (B, N), 1)
    masked0 = jnp.where(conf > CONFIDENCE_THRESHOLD, conf, NEG_INF)

    pt = jnp.concatenate(
        [jnp.transpose(x1), jnp.transpose(y1),
         jnp.transpose(w), jnp.transpose(h)], axis=1)
    pt_hi = pt.astype(jnp.bfloat16)
    ptr = pt - pt_hi.astype(jnp.float32)
    pt_mid = ptr.astype(jnp.bfloat16)
    pt_lo = (ptr - pt_mid.astype(jnp.float32)).astype(jnp.bfloat16)
    lane32 = lax.broadcasted_iota(jnp.int32, (B, 4 * B), 1)
    sub32 = lax.broadcasted_iota(jnp.int32, (B, 4 * B), 0)
    diag = (lane32 % B) == sub32

    def step(t, masked):
        maxval = jnp.max(masked, axis=1, keepdims=True)
        elig = masked == maxval
        idx = jnp.min(jnp.where(elig, iota, N), axis=1, keepdims=True)
        onehot = iota == idx
        valid = maxval > CONFIDENCE_THRESHOLD
        v = valid.astype(jnp.float32)

        oh_bf = onehot.astype(jnp.bfloat16)
        selmat = (jnp.dot(oh_bf, pt_hi, preferred_element_type=jnp.float32)
                  + jnp.dot(oh_bf, pt_mid, preferred_element_type=jnp.float32)
                  + jnp.dot(oh_bf, pt_lo, preferred_element_type=jnp.float32))
        dm = jnp.where(diag, selmat, 0.0)
        sx1 = jnp.sum(dm[:, 0 * B:1 * B], axis=1, keepdims=True)
        sy1 = jnp.sum(dm[:, 1 * B:2 * B], axis=1, keepdims=True)
        sw = jnp.sum(dm[:, 2 * B:3 * B], axis=1, keepdims=True)
        sh = jnp.sum(dm[:, 3 * B:4 * B], axis=1, keepdims=True)
        samin = jnp.minimum(sx1, sw)
        samax = jnp.maximum(sx1, sw)
        sbmin = jnp.minimum(sy1, sh)
        sbmax = jnp.maximum(sy1, sh)
        sarea = (samax - samin) * (sbmax - sbmin)

        inter_a = jnp.maximum(0.0, jnp.minimum(samax, a_max) - jnp.maximum(samin, a_min))
        inter_b = jnp.maximum(0.0, jnp.minimum(sbmax, b_max) - jnp.maximum(sbmin, b_min))
        inter = inter_a * inter_b
        union = sarea + areas - inter
        denom = jnp.where(union > 0.0, union, 1.0)
        iou = jnp.where(union > 0.0, inter / denom, 0.0)
        suppress = iou > IOU_THRESHOLD

        new_masked = jnp.where(suppress | onehot, NEG_INF, masked)

        obx_ref[pl.ds(t, 1), :] = (sx1 * v).reshape(1, B)
        oby_ref[pl.ds(t, 1), :] = (sy1 * v).reshape(1, B)
        obw_ref[pl.ds(t, 1), :] = (sw * v).reshape(1, B)
        obh_ref[pl.ds(t, 1), :] = (sh * v).reshape(1, B)
        oconf_ref[pl.ds(t, 1), :] = jnp.where(valid, maxval, 0.0).reshape(1, B)
        idx_scr[pl.ds(t, 1), :] = idx.astype(jnp.int32).reshape(1, B)
        val_scr[pl.ds(t, 1), :] = v.reshape(1, B)
        return new_masked

    lax.fori_loop(0, MAX_DETECTIONS, step, masked0, unroll=4)

    iota_n = lax.broadcasted_iota(jnp.int32, (MAX_DETECTIONS, N), 1)
    for b in range(B):
        idx_b = idx_scr[:, b].reshape(MAX_DETECTIONS, 1)
        onehot_b = (iota_n == idx_b).astype(jnp.bfloat16)
        lg = logits_ref[b]
        hi = lg.astype(jnp.bfloat16)
        r1 = lg - hi.astype(jnp.float32)
        mid = r1.astype(jnp.bfloat16)
        lo = (r1 - mid.astype(jnp.float32)).astype(jnp.bfloat16)
        rows = (jnp.dot(onehot_b, hi, preferred_element_type=jnp.float32)
                + jnp.dot(onehot_b, mid, preferred_element_type=jnp.float32)
                + jnp.dot(onehot_b, lo, preferred_element_type=jnp.float32))
        m = jnp.max(rows, axis=1, keepdims=True)
        e = jnp.exp(rows - m)
        p = e / jnp.sum(e, axis=1, keepdims=True)
        ocls_ref[b] = p * val_scr[:, b].reshape(MAX_DETECTIONS, 1)


@functools.partial(jax.jit, static_argnames=("interpret",))
def kernel(box_prediction, class_prediction, interpret=False):
    B, N, C = class_prediction.shape
    x1 = box_prediction[..., 0]
    y1 = box_prediction[..., 1]
    x2 = box_prediction[..., 2]
    y2 = box_prediction[..., 3]

    out_shapes = (
        jax.ShapeDtypeStruct((MAX_DETECTIONS, B), jnp.float32),
        jax.ShapeDtypeStruct((MAX_DETECTIONS, B), jnp.float32),
        jax.ShapeDtypeStruct((MAX_DETECTIONS, B), jnp.float32),
        jax.ShapeDtypeStruct((MAX_DETECTIONS, B), jnp.float32),
        jax.ShapeDtypeStruct((MAX_DETECTIONS, B), jnp.float32),
        jax.ShapeDtypeStruct((B, MAX_DETECTIONS, C), jnp.float32),
    )
    bx, by, bw, bh, cf, cls = pl.pallas_call(
        _nms_body,
        out_shape=out_shapes,
        scratch_shapes=[
            pltpu.VMEM((MAX_DETECTIONS, B), jnp.int32),
            pltpu.VMEM((MAX_DETECTIONS, B), jnp.float32),
        ],
        interpret=interpret,
    )(x1, y1, x2, y2, class_prediction)

    nms_box = jnp.stack([bx.T, by.T, bw.T, bh.T], axis=-1)
    nms_conf = cf.T
    return nms_box, cls, nms_conf


# speedup vs baseline: 1.5863x; 1.5863x over previous
import functools

import jax
import jax.numpy as jnp
from jax import lax
from jax.experimental import pallas as pl
from jax.experimental.pallas import tpu as pltpu

IOU_THRESHOLD = 0.5
CONFIDENCE_THRESHOLD = 0.5
MAX_DETECTIONS = 100
NEG_INF = float("-inf")


def _nms_body(x1_ref, y1_ref, x2_ref, y2_ref, logits_ref,
              obx_ref, oby_ref, obw_ref, obh_ref, oconf_ref, ocls_ref,
              idx_scr, val_scr):
    B, N = x1_ref.shape
    x1 = x1_ref[...]
    y1 = y1_ref[...]
    w = x2_ref[...] - x1
    h = y2_ref[...] - y1
    a_min = jnp.minimum(x1, w)
    a_max = jnp.maximum(x1, w)
    b_min = jnp.minimum(y1, h)
    b_max = jnp.maximum(y1, h)
    areas = (a_max - a_min) * (b_max - b_min)

    parts = []
    for b in range(B):
        red = jnp.max(logits_ref[b], axis=-1, keepdims=True)
        parts.append(jnp.transpose(red, (1, 0)))
    conf = jax.nn.sigmoid(jnp.concatenate(parts, axis=0))

    iota = lax.broadcasted_iota(jnp.int32, (B, N), 1)
    masked0 = jnp.where(conf > CONFIDENCE_THRESHOLD, conf, NEG_INF)

    def step(t, masked):
        maxval = jnp.max(masked, axis=1, keepdims=True)
        elig = masked == maxval
        idx = jnp.min(jnp.where(elig, iota, N), axis=1, keepdims=True)
        onehot = iota == idx
        valid = maxval > CONFIDENCE_THRESHOLD
        v = valid.astype(jnp.float32)

        def sel(arr):
            return jnp.sum(jnp.where(onehot, arr, 0.0), axis=1, keepdims=True)

        sx1 = sel(x1)
        sy1 = sel(y1)
        sw = sel(w)
        sh = sel(h)
        samin = jnp.minimum(sx1, sw)
        samax = jnp.maximum(sx1, sw)
        sbmin = jnp.minimum(sy1, sh)
        sbmax = jnp.maximum(sy1, sh)
        sarea = (samax - samin) * (sbmax - sbmin)

        inter_a = jnp.maximum(0.0, jnp.minimum(samax, a_max) - jnp.maximum(samin, a_min))
        inter_b = jnp.maximum(0.0, jnp.minimum(sbmax, b_max) - jnp.maximum(sbmin, b_min))
        inter = inter_a * inter_b
        union = sarea + areas - inter
        denom = jnp.where(union > 0.0, union, 1.0)
        iou = jnp.where(union > 0.0, inter / denom, 0.0)
        suppress = iou > IOU_THRESHOLD

        new_masked = jnp.where(suppress | onehot, NEG_INF, masked)

        obx_ref[pl.ds(t, 1), :] = (sx1 * v).reshape(1, B)
        oby_ref[pl.ds(t, 1), :] = (sy1 * v).reshape(1, B)
        obw_ref[pl.ds(t, 1), :] = (sw * v).reshape(1, B)
        obh_ref[pl.ds(t, 1), :] = (sh * v).reshape(1, B)
        oconf_ref[pl.ds(t, 1), :] = jnp.where(valid, maxval, 0.0).reshape(1, B)
        idx_scr[pl.ds(t, 1), :] = idx.astype(jnp.int32).reshape(1, B)
        val_scr[pl.ds(t, 1), :] = v.reshape(1, B)
        return new_masked

    lax.fori_loop(0, MAX_DETECTIONS, step, masked0, unroll=4)

    iota_n = lax.broadcasted_iota(jnp.int32, (MAX_DETECTIONS, N), 1)
    for b in range(B):
        idx_b = idx_scr[:, b].reshape(MAX_DETECTIONS, 1)
        onehot_b = (iota_n == idx_b).astype(jnp.bfloat16)
        lg = logits_ref[b]
        hi = lg.astype(jnp.bfloat16)
        r1 = lg - hi.astype(jnp.float32)
        mid = r1.astype(jnp.bfloat16)
        lo = (r1 - mid.astype(jnp.float32)).astype(jnp.bfloat16)
        rows = (jnp.dot(onehot_b, hi, preferred_element_type=jnp.float32)
                + jnp.dot(onehot_b, mid, preferred_element_type=jnp.float32)
                + jnp.dot(onehot_b, lo, preferred_element_type=jnp.float32))
        m = jnp.max(rows, axis=1, keepdims=True)
        e = jnp.exp(rows - m)
        p = e / jnp.sum(e, axis=1, keepdims=True)
        ocls_ref[b] = p * val_scr[:, b].reshape(MAX_DETECTIONS, 1)


@functools.partial(jax.jit, static_argnames=("interpret",))
def kernel(box_prediction, class_prediction, interpret=False):
    B, N, C = class_prediction.shape
    x1 = box_prediction[..., 0]
    y1 = box_prediction[..., 1]
    x2 = box_prediction[..., 2]
    y2 = box_prediction[..., 3]

    out_shapes = (
        jax.ShapeDtypeStruct((MAX_DETECTIONS, B), jnp.float32),
        jax.ShapeDtypeStruct((MAX_DETECTIONS, B), jnp.float32),
        jax.ShapeDtypeStruct((MAX_DETECTIONS, B), jnp.float32),
        jax.ShapeDtypeStruct((MAX_DETECTIONS, B), jnp.float32),
        jax.ShapeDtypeStruct((MAX_DETECTIONS, B), jnp.float32),
        jax.ShapeDtypeStruct((B, MAX_DETECTIONS, C), jnp.float32),
    )
    bx, by, bw, bh, cf, cls = pl.pallas_call(
        _nms_body,
        out_shape=out_shapes,
        scratch_shapes=[
            pltpu.VMEM((MAX_DETECTIONS, B), jnp.int32),
            pltpu.VMEM((MAX_DETECTIONS, B), jnp.float32),
        ],
        interpret=interpret,
    )(x1, y1, x2, y2, class_prediction)

    nms_box = jnp.stack([bx.T, by.T, bw.T, bh.T], axis=-1)
    nms_conf = cf.T
    return nms_box, cls, nms_conf


# speedup vs baseline: 1.6479x; 1.0388x over previous
import functools

import jax
import jax.numpy as jnp
from jax import lax
from jax.experimental import pallas as pl
from jax.experimental.pallas import tpu as pltpu

IOU_THRESHOLD = 0.5
CONFIDENCE_THRESHOLD = 0.5
MAX_DETECTIONS = 100
NEG_INF = float("-inf")


def _nms_body(x1_ref, y1_ref, x2_ref, y2_ref, logits_ref,
              obx_ref, oby_ref, obw_ref, obh_ref, oconf_ref, ocls_ref,
              idx_scr, val_scr):
    B, N = x1_ref.shape
    x1 = x1_ref[...]
    y1 = y1_ref[...]
    w = x2_ref[...] - x1
    h = y2_ref[...] - y1
    a_min = jnp.minimum(x1, w)
    a_max = jnp.maximum(x1, w)
    b_min = jnp.minimum(y1, h)
    b_max = jnp.maximum(y1, h)
    areas = (a_max - a_min) * (b_max - b_min)

    conf = jax.nn.sigmoid(jnp.max(logits_ref[...], axis=-1))

    iota = lax.broadcasted_iota(jnp.int32, (B, N), 1)
    masked0 = jnp.where(conf > CONFIDENCE_THRESHOLD, conf, NEG_INF)

    def step(t, masked):
        maxval = jnp.max(masked, axis=1, keepdims=True)
        elig = masked == maxval
        idx = jnp.min(jnp.where(elig, iota, N), axis=1, keepdims=True)
        onehot = iota == idx
        valid = maxval > CONFIDENCE_THRESHOLD
        v = valid.astype(jnp.float32)

        def sel(arr):
            return jnp.sum(jnp.where(onehot, arr, 0.0), axis=1, keepdims=True)

        sx1 = sel(x1)
        sy1 = sel(y1)
        sw = sel(w)
        sh = sel(h)
        samin = jnp.minimum(sx1, sw)
        samax = jnp.maximum(sx1, sw)
        sbmin = jnp.minimum(sy1, sh)
        sbmax = jnp.maximum(sy1, sh)
        sarea = (samax - samin) * (sbmax - sbmin)

        inter_a = jnp.maximum(0.0, jnp.minimum(samax, a_max) - jnp.maximum(samin, a_min))
        inter_b = jnp.maximum(0.0, jnp.minimum(sbmax, b_max) - jnp.maximum(sbmin, b_min))
        inter = inter_a * inter_b
        union = sarea + areas - inter
        denom = jnp.where(union > 0.0, union, 1.0)
        iou = jnp.where(union > 0.0, inter / denom, 0.0)
        suppress = iou > IOU_THRESHOLD

        new_masked = jnp.where(suppress | onehot, NEG_INF, masked)

        obx_ref[pl.ds(t, 1), :] = (sx1 * v).reshape(1, B)
        oby_ref[pl.ds(t, 1), :] = (sy1 * v).reshape(1, B)
        obw_ref[pl.ds(t, 1), :] = (sw * v).reshape(1, B)
        obh_ref[pl.ds(t, 1), :] = (sh * v).reshape(1, B)
        oconf_ref[pl.ds(t, 1), :] = jnp.where(valid, maxval, 0.0).reshape(1, B)
        idx_scr[pl.ds(t, 1), :] = idx.astype(jnp.int32).reshape(1, B)
        val_scr[pl.ds(t, 1), :] = v.reshape(1, B)
        return new_masked

    lax.fori_loop(0, MAX_DETECTIONS, step, masked0, unroll=4)

    iota_n = lax.broadcasted_iota(jnp.int32, (MAX_DETECTIONS, N), 1)
    for b in range(B):
        idx_b = idx_scr[:, b].reshape(MAX_DETECTIONS, 1)
        onehot_b = (iota_n == idx_b).astype(jnp.bfloat16)
        lg = logits_ref[b]
        hi = lg.astype(jnp.bfloat16)
        r1 = lg - hi.astype(jnp.float32)
        mid = r1.astype(jnp.bfloat16)
        lo = (r1 - mid.astype(jnp.float32)).astype(jnp.bfloat16)
        rows = (jnp.dot(onehot_b, hi, preferred_element_type=jnp.float32)
                + jnp.dot(onehot_b, mid, preferred_element_type=jnp.float32)
                + jnp.dot(onehot_b, lo, preferred_element_type=jnp.float32))
        m = jnp.max(rows, axis=1, keepdims=True)
        e = jnp.exp(rows - m)
        p = e / jnp.sum(e, axis=1, keepdims=True)
        ocls_ref[b] = p * val_scr[:, b].reshape(MAX_DETECTIONS, 1)


@functools.partial(jax.jit, static_argnames=("interpret",))
def kernel(box_prediction, class_prediction, interpret=False):
    B, N, C = class_prediction.shape
    x1 = box_prediction[..., 0]
    y1 = box_prediction[..., 1]
    x2 = box_prediction[..., 2]
    y2 = box_prediction[..., 3]

    out_shapes = (
        jax.ShapeDtypeStruct((MAX_DETECTIONS, B), jnp.float32),
        jax.ShapeDtypeStruct((MAX_DETECTIONS, B), jnp.float32),
        jax.ShapeDtypeStruct((MAX_DETECTIONS, B), jnp.float32),
        jax.ShapeDtypeStruct((MAX_DETECTIONS, B), jnp.float32),
        jax.ShapeDtypeStruct((MAX_DETECTIONS, B), jnp.float32),
        jax.ShapeDtypeStruct((B, MAX_DETECTIONS, C), jnp.float32),
    )
    bx, by, bw, bh, cf, cls = pl.pallas_call(
        _nms_body,
        out_shape=out_shapes,
        scratch_shapes=[
            pltpu.VMEM((MAX_DETECTIONS, B), jnp.int32),
            pltpu.VMEM((MAX_DETECTIONS, B), jnp.float32),
        ],
        interpret=interpret,
    )(x1, y1, x2, y2, class_prediction)

    nms_box = jnp.stack([bx.T, by.T, bw.T, bh.T], axis=-1)
    nms_conf = cf.T
    return nms_box, cls, nms_conf


# speedup vs baseline: 1.7301x; 1.0499x over previous
import functools

import jax
import jax.numpy as jnp
from jax import lax
from jax.experimental import pallas as pl
from jax.experimental.pallas import tpu as pltpu

IOU_THRESHOLD = 0.5
CONFIDENCE_THRESHOLD = 0.5
MAX_DETECTIONS = 100
NEG_INF = float("-inf")


def _nms_body(x1_ref, y1_ref, x2_ref, y2_ref, logits_ref,
              obx_ref, oby_ref, obw_ref, obh_ref, oconf_ref, ocls_ref,
              idx_scr, val_scr):
    B, NB, NL = x1_ref.shape
    N = logits_ref.shape[1]
    x1 = x1_ref[...]
    y1 = y1_ref[...]
    w = x2_ref[...] - x1
    h = y2_ref[...] - y1
    a_min = jnp.minimum(x1, w)
    a_max = jnp.maximum(x1, w)
    b_min = jnp.minimum(y1, h)
    b_max = jnp.maximum(y1, h)
    areas = (a_max - a_min) * (b_max - b_min)

    conf = jax.nn.sigmoid(jnp.max(logits_ref[...], axis=-1))

    iota = (lax.broadcasted_iota(jnp.int32, (B, NB, NL), 1) * NL
            + lax.broadcasted_iota(jnp.int32, (B, NB, NL), 2))
    masked0 = jnp.reshape(
        jnp.concatenate(
            [jnp.where(conf > CONFIDENCE_THRESHOLD, conf, NEG_INF),
             jnp.full((B, NB * NL - N), NEG_INF, jnp.float32)], axis=1),
        (B, NB, NL))

    def step(t, masked):
        maxval = jnp.max(masked, axis=(1, 2), keepdims=True)
        elig = masked == maxval
        idx = jnp.min(jnp.where(elig, iota, NB * NL),
                      axis=(1, 2), keepdims=True)
        onehot = iota == idx
        valid = maxval > CONFIDENCE_THRESHOLD
        v = valid.astype(jnp.float32)

        def sel(arr):
            return jnp.sum(jnp.where(onehot, arr, 0.0),
                           axis=(1, 2), keepdims=True)

        sx1 = sel(x1)
        sy1 = sel(y1)
        sw = sel(w)
        sh = sel(h)
        samin = jnp.minimum(sx1, sw)
        samax = jnp.maximum(sx1, sw)
        sbmin = jnp.minimum(sy1, sh)
        sbmax = jnp.maximum(sy1, sh)
        sarea = (samax - samin) * (sbmax - sbmin)

        inter_a = jnp.maximum(0.0, jnp.minimum(samax, a_max) - jnp.maximum(samin, a_min))
        inter_b = jnp.maximum(0.0, jnp.minimum(sbmax, b_max) - jnp.maximum(sbmin, b_min))
        inter = inter_a * inter_b
        union = sarea + areas - inter
        denom = jnp.where(union > 0.0, union, 1.0)
        iou = jnp.where(union > 0.0, inter / denom, 0.0)
        suppress = iou > IOU_THRESHOLD

        new_masked = jnp.where(suppress | onehot, NEG_INF, masked)

        obx_ref[pl.ds(t, 1), :] = (sx1 * v).reshape(1, B)
        oby_ref[pl.ds(t, 1), :] = (sy1 * v).reshape(1, B)
        obw_ref[pl.ds(t, 1), :] = (sw * v).reshape(1, B)
        obh_ref[pl.ds(t, 1), :] = (sh * v).reshape(1, B)
        oconf_ref[pl.ds(t, 1), :] = jnp.where(valid, maxval, 0.0).reshape(1, B)
        idx_scr[pl.ds(t, 1), :] = idx.astype(jnp.int32).reshape(1, B)
        val_scr[pl.ds(t, 1), :] = v.reshape(1, B)
        return new_masked

    lax.fori_loop(0, MAX_DETECTIONS, step, masked0, unroll=4)

    iota_n = lax.broadcasted_iota(jnp.int32, (MAX_DETECTIONS, N), 1)
    for b in range(B):
        idx_b = idx_scr[:, b].reshape(MAX_DETECTIONS, 1)
        onehot_b = (iota_n == idx_b).astype(jnp.bfloat16)
        lg = logits_ref[b]
        hi = lg.astype(jnp.bfloat16)
        r1 = lg - hi.astype(jnp.float32)
        mid = r1.astype(jnp.bfloat16)
        lo = (r1 - mid.astype(jnp.float32)).astype(jnp.bfloat16)
        rows = (jnp.dot(onehot_b, hi, preferred_element_type=jnp.float32)
                + jnp.dot(onehot_b, mid, preferred_element_type=jnp.float32)
                + jnp.dot(onehot_b, lo, preferred_element_type=jnp.float32))
        m = jnp.max(rows, axis=1, keepdims=True)
        e = jnp.exp(rows - m)
        p = e / jnp.sum(e, axis=1, keepdims=True)
        ocls_ref[b] = p * val_scr[:, b].reshape(MAX_DETECTIONS, 1)


@functools.partial(jax.jit, static_argnames=("interpret",))
def kernel(box_prediction, class_prediction, interpret=False):
    B, N, C = class_prediction.shape
    NB, NL = 40, 128
    padw = ((0, 0), (0, NB * NL - N))

    def prep(a):
        return jnp.pad(a, padw).reshape(B, NB, NL)

    x1 = prep(box_prediction[..., 0])
    y1 = prep(box_prediction[..., 1])
    x2 = prep(box_prediction[..., 2])
    y2 = prep(box_prediction[..., 3])

    out_shapes = (
        jax.ShapeDtypeStruct((MAX_DETECTIONS, B), jnp.float32),
        jax.ShapeDtypeStruct((MAX_DETECTIONS, B), jnp.float32),
        jax.ShapeDtypeStruct((MAX_DETECTIONS, B), jnp.float32),
        jax.ShapeDtypeStruct((MAX_DETECTIONS, B), jnp.float32),
        jax.ShapeDtypeStruct((MAX_DETECTIONS, B), jnp.float32),
        jax.ShapeDtypeStruct((B, MAX_DETECTIONS, C), jnp.float32),
    )
    bx, by, bw, bh, cf, cls = pl.pallas_call(
        _nms_body,
        out_shape=out_shapes,
        scratch_shapes=[
            pltpu.VMEM((MAX_DETECTIONS, B), jnp.int32),
            pltpu.VMEM((MAX_DETECTIONS, B), jnp.float32),
        ],
        interpret=interpret,
    )(x1, y1, x2, y2, class_prediction)

    nms_box = jnp.stack([bx.T, by.T, bw.T, bh.T], axis=-1)
    nms_conf = cf.T
    return nms_box, cls, nms_conf


# speedup vs baseline: 1.7449x; 1.0086x over previous
import functools

import jax
import jax.numpy as jnp
from jax import lax
from jax.experimental import pallas as pl
from jax.experimental.pallas import tpu as pltpu

IOU_THRESHOLD = 0.5
CONFIDENCE_THRESHOLD = 0.5
MAX_DETECTIONS = 100
NEG_INF = float("-inf")


def _nms_body(x1_ref, y1_ref, x2_ref, y2_ref, logits_ref,
              obx_ref, oby_ref, obw_ref, obh_ref, oconf_ref, ocls_ref,
              idx_scr, val_scr):
    B, NB, NL = x1_ref.shape
    N = logits_ref.shape[1]
    x1 = x1_ref[...]
    y1 = y1_ref[...]
    w = x2_ref[...] - x1
    h = y2_ref[...] - y1
    a_min = jnp.minimum(x1, w)
    a_max = jnp.maximum(x1, w)
    b_min = jnp.minimum(y1, h)
    b_max = jnp.maximum(y1, h)
    areas = (a_max - a_min) * (b_max - b_min)

    conf = jax.nn.sigmoid(jnp.max(logits_ref[...], axis=-1))

    iota = (lax.broadcasted_iota(jnp.int32, (B, NB, NL), 1) * NL
            + lax.broadcasted_iota(jnp.int32, (B, NB, NL), 2))
    masked0 = jnp.reshape(
        jnp.concatenate(
            [jnp.where(conf > CONFIDENCE_THRESHOLD, conf, NEG_INF),
             jnp.full((B, NB * NL - N), NEG_INF, jnp.float32)], axis=1),
        (B, NB, NL))

    def step(t, masked):
        maxval = jnp.max(masked, axis=(1, 2), keepdims=True)
        elig = masked == maxval
        idx = jnp.min(jnp.where(elig, iota, NB * NL),
                      axis=(1, 2), keepdims=True)
        onehot = iota == idx
        valid = maxval > CONFIDENCE_THRESHOLD
        v = valid.astype(jnp.float32)

        def sel(arr):
            return jnp.sum(jnp.where(onehot, arr, 0.0),
                           axis=(1, 2), keepdims=True)

        sx1 = sel(x1)
        sy1 = sel(y1)
        sw = sel(w)
        sh = sel(h)
        samin = jnp.minimum(sx1, sw)
        samax = jnp.maximum(sx1, sw)
        sbmin = jnp.minimum(sy1, sh)
        sbmax = jnp.maximum(sy1, sh)
        sarea = (samax - samin) * (sbmax - sbmin)

        inter_a = jnp.maximum(0.0, jnp.minimum(samax, a_max) - jnp.maximum(samin, a_min))
        inter_b = jnp.maximum(0.0, jnp.minimum(sbmax, b_max) - jnp.maximum(sbmin, b_min))
        inter = inter_a * inter_b
        union = sarea + areas - inter
        denom = jnp.where(union > 0.0, union, 1.0)
        iou = jnp.where(union > 0.0, inter / denom, 0.0)
        suppress = iou > IOU_THRESHOLD

        new_masked = jnp.where(suppress | onehot, NEG_INF, masked)

        obx_ref[pl.ds(t, 1), :] = (sx1 * v).reshape(1, B)
        oby_ref[pl.ds(t, 1), :] = (sy1 * v).reshape(1, B)
        obw_ref[pl.ds(t, 1), :] = (sw * v).reshape(1, B)
        obh_ref[pl.ds(t, 1), :] = (sh * v).reshape(1, B)
        oconf_ref[pl.ds(t, 1), :] = jnp.where(valid, maxval, 0.0).reshape(1, B)
        idx_scr[pl.ds(t, 1), :] = idx.astype(jnp.int32).reshape(1, B)
        val_scr[pl.ds(t, 1), :] = v.reshape(1, B)
        return new_masked

    lax.fori_loop(0, MAX_DETECTIONS, step, masked0, unroll=8)

    iota_n = lax.broadcasted_iota(jnp.int32, (MAX_DETECTIONS, N), 1)
    for b in range(B):
        idx_b = idx_scr[:, b].reshape(MAX_DETECTIONS, 1)
        onehot_b = (iota_n == idx_b).astype(jnp.bfloat16)
        lg = logits_ref[b]
        hi = lg.astype(jnp.bfloat16)
        r1 = lg - hi.astype(jnp.float32)
        mid = r1.astype(jnp.bfloat16)
        lo = (r1 - mid.astype(jnp.float32)).astype(jnp.bfloat16)
        rows = (jnp.dot(onehot_b, hi, preferred_element_type=jnp.float32)
                + jnp.dot(onehot_b, mid, preferred_element_type=jnp.float32)
                + jnp.dot(onehot_b, lo, preferred_element_type=jnp.float32))
        m = jnp.max(rows, axis=1, keepdims=True)
        e = jnp.exp(rows - m)
        p = e / jnp.sum(e, axis=1, keepdims=True)
        ocls_ref[b] = p * val_scr[:, b].reshape(MAX_DETECTIONS, 1)


@functools.partial(jax.jit, static_argnames=("interpret",))
def kernel(box_prediction, class_prediction, interpret=False):
    B, N, C = class_prediction.shape
    NB, NL = 40, 128
    padw = ((0, 0), (0, NB * NL - N))

    def prep(a):
        return jnp.pad(a, padw).reshape(B, NB, NL)

    x1 = prep(box_prediction[..., 0])
    y1 = prep(box_prediction[..., 1])
    x2 = prep(box_prediction[..., 2])
    y2 = prep(box_prediction[..., 3])

    out_shapes = (
        jax.ShapeDtypeStruct((MAX_DETECTIONS, B), jnp.float32),
        jax.ShapeDtypeStruct((MAX_DETECTIONS, B), jnp.float32),
        jax.ShapeDtypeStruct((MAX_DETECTIONS, B), jnp.float32),
        jax.ShapeDtypeStruct((MAX_DETECTIONS, B), jnp.float32),
        jax.ShapeDtypeStruct((MAX_DETECTIONS, B), jnp.float32),
        jax.ShapeDtypeStruct((B, MAX_DETECTIONS, C), jnp.float32),
    )
    bx, by, bw, bh, cf, cls = pl.pallas_call(
        _nms_body,
        out_shape=out_shapes,
        scratch_shapes=[
            pltpu.VMEM((MAX_DETECTIONS, B), jnp.int32),
            pltpu.VMEM((MAX_DETECTIONS, B), jnp.float32),
        ],
        interpret=interpret,
    )(x1, y1, x2, y2, class_prediction)

    nms_box = jnp.stack([bx.T, by.T, bw.T, bh.T], axis=-1)
    nms_conf = cf.T
    return nms_box, cls, nms_conf


# speedup vs baseline: 1.7455x; 1.0004x over previous
import functools

import jax
import jax.numpy as jnp
from jax import lax
from jax.experimental import pallas as pl
from jax.experimental.pallas import tpu as pltpu

IOU_THRESHOLD = 0.5
CONFIDENCE_THRESHOLD = 0.5
MAX_DETECTIONS = 100
NEG_INF = float("-inf")


def _nms_body(x1_ref, y1_ref, x2_ref, y2_ref, logits_ref,
              obx_ref, oby_ref, obw_ref, obh_ref, oconf_ref, ocls_ref,
              idx_scr, val_scr):
    B, NB, NL = x1_ref.shape
    N = logits_ref.shape[1]
    x1 = x1_ref[...]
    y1 = y1_ref[...]
    w = x2_ref[...] - x1
    h = y2_ref[...] - y1
    a_min = jnp.minimum(x1, w)
    a_max = jnp.maximum(x1, w)
    b_min = jnp.minimum(y1, h)
    b_max = jnp.maximum(y1, h)
    areas = (a_max - a_min) * (b_max - b_min)

    conf = jax.nn.sigmoid(jnp.max(logits_ref[...], axis=-1))

    iota = (lax.broadcasted_iota(jnp.int32, (B, NB, NL), 1) * NL
            + lax.broadcasted_iota(jnp.int32, (B, NB, NL), 2))
    masked0 = jnp.reshape(
        jnp.concatenate(
            [jnp.where(conf > CONFIDENCE_THRESHOLD, conf, NEG_INF),
             jnp.full((B, NB * NL - N), NEG_INF, jnp.float32)], axis=1),
        (B, NB, NL))

    def step(t, masked):
        maxval = jnp.max(masked, axis=(1, 2), keepdims=True)
        elig = masked == maxval
        idx = jnp.min(jnp.where(elig, iota, NB * NL),
                      axis=(1, 2), keepdims=True)
        onehot = iota == idx
        valid = maxval > CONFIDENCE_THRESHOLD
        v = valid.astype(jnp.float32)

        def sel(arr):
            return jnp.sum(jnp.where(onehot, arr, 0.0),
                           axis=(1, 2), keepdims=True)

        sx1 = sel(x1)
        sy1 = sel(y1)
        sw = sel(w)
        sh = sel(h)
        samin = jnp.minimum(sx1, sw)
        samax = jnp.maximum(sx1, sw)
        sbmin = jnp.minimum(sy1, sh)
        sbmax = jnp.maximum(sy1, sh)
        sarea = (samax - samin) * (sbmax - sbmin)

        inter_a = jnp.maximum(0.0, jnp.minimum(samax, a_max) - jnp.maximum(samin, a_min))
        inter_b = jnp.maximum(0.0, jnp.minimum(sbmax, b_max) - jnp.maximum(sbmin, b_min))
        inter = inter_a * inter_b
        union = sarea + areas - inter
        denom = jnp.where(union > 0.0, union, 1.0)
        iou = jnp.where(union > 0.0, inter / denom, 0.0)
        suppress = iou > IOU_THRESHOLD

        new_masked = jnp.where(suppress | onehot, NEG_INF, masked)

        obx_ref[pl.ds(t, 1), :] = (sx1 * v).reshape(1, B)
        oby_ref[pl.ds(t, 1), :] = (sy1 * v).reshape(1, B)
        obw_ref[pl.ds(t, 1), :] = (sw * v).reshape(1, B)
        obh_ref[pl.ds(t, 1), :] = (sh * v).reshape(1, B)
        oconf_ref[pl.ds(t, 1), :] = jnp.where(valid, maxval, 0.0).reshape(1, B)
        idx_scr[pl.ds(t, 1), :] = idx.astype(jnp.int32).reshape(1, B)
        val_scr[pl.ds(t, 1), :] = v.reshape(1, B)
        return new_masked

    lax.fori_loop(0, MAX_DETECTIONS, step, masked0, unroll=10)

    iota_n = lax.broadcasted_iota(jnp.int32, (MAX_DETECTIONS, N), 1)
    for b in range(B):
        idx_b = idx_scr[:, b].reshape(MAX_DETECTIONS, 1)
        onehot_b = (iota_n == idx_b).astype(jnp.bfloat16)
        lg = logits_ref[b]
        hi = lg.astype(jnp.bfloat16)
        r1 = lg - hi.astype(jnp.float32)
        mid = r1.astype(jnp.bfloat16)
        lo = (r1 - mid.astype(jnp.float32)).astype(jnp.bfloat16)
        rows = (jnp.dot(onehot_b, hi, preferred_element_type=jnp.float32)
                + jnp.dot(onehot_b, mid, preferred_element_type=jnp.float32)
                + jnp.dot(onehot_b, lo, preferred_element_type=jnp.float32))
        m = jnp.max(rows, axis=1, keepdims=True)
        e = jnp.exp(rows - m)
        p = e / jnp.sum(e, axis=1, keepdims=True)
        ocls_ref[b] = p * val_scr[:, b].reshape(MAX_DETECTIONS, 1)


@functools.partial(jax.jit, static_argnames=("interpret",))
def kernel(box_prediction, class_prediction, interpret=False):
    B, N, C = class_prediction.shape
    NB, NL = 40, 128
    padw = ((0, 0), (0, NB * NL - N))

    def prep(a):
        return jnp.pad(a, padw).reshape(B, NB, NL)

    x1 = prep(box_prediction[..., 0])
    y1 = prep(box_prediction[..., 1])
    x2 = prep(box_prediction[..., 2])
    y2 = prep(box_prediction[..., 3])

    out_shapes = (
        jax.ShapeDtypeStruct((MAX_DETECTIONS, B), jnp.float32),
        jax.ShapeDtypeStruct((MAX_DETECTIONS, B), jnp.float32),
        jax.ShapeDtypeStruct((MAX_DETECTIONS, B), jnp.float32),
        jax.ShapeDtypeStruct((MAX_DETECTIONS, B), jnp.float32),
        jax.ShapeDtypeStruct((MAX_DETECTIONS, B), jnp.float32),
        jax.ShapeDtypeStruct((B, MAX_DETECTIONS, C), jnp.float32),
    )
    bx, by, bw, bh, cf, cls = pl.pallas_call(
        _nms_body,
        out_shape=out_shapes,
        scratch_shapes=[
            pltpu.VMEM((MAX_DETECTIONS, B), jnp.int32),
            pltpu.VMEM((MAX_DETECTIONS, B), jnp.float32),
        ],
        interpret=interpret,
    )(x1, y1, x2, y2, class_prediction)

    nms_box = jnp.stack([bx.T, by.T, bw.T, bh.T], axis=-1)
    nms_conf = cf.T
    return nms_box, cls, nms_conf
